# Initial kernel scaffold; baseline (speedup 1.0000x reference)
import jax
import jax.numpy as jnp
from jax.experimental import pallas as pl


def kernel(orig, aug, l, adj):
    raise NotImplementedError("write your pallas kernel here")


# speedup vs baseline: 115.9769x; 115.9769x over previous
import jax
import jax.numpy as jnp
from jax.experimental import pallas as pl
from jax.experimental.pallas import tpu as pltpu

_N, _D = 2048, 128
_TR = 256
_MARGIN = 1.0
_EPS = 1e-6


def _loss_body(orig_ref, anch_ref, aug_ref, lc_ref, lr_ref, adj_ref, out_ref):
    i = pl.program_id(0)
    o = orig_ref[...]
    g = aug_ref[...]
    a = anch_ref[...]

    no_i = jnp.sum(a * a, axis=1, keepdims=True)
    so_i = jnp.sum(a, axis=1, keepdims=True)

    ones = jnp.ones((1, _D), jnp.float32)
    dn = (((1,), (1,)), ((), ()))
    no_j = jax.lax.dot_general(ones, o * o, dn, preferred_element_type=jnp.float32)
    so_j = jax.lax.dot_general(ones, o, dn, preferred_element_type=jnp.float32)
    na_j = jax.lax.dot_general(ones, g * g, dn, preferred_element_type=jnp.float32)
    sa_j = jax.lax.dot_general(ones, g, dn, preferred_element_type=jnp.float32)

    gp = jax.lax.dot_general(a, o, dn, preferred_element_type=jnp.float32,
                             precision=jax.lax.Precision.HIGHEST)
    gn = jax.lax.dot_general(a, g, dn, preferred_element_type=jnp.float32,
                             precision=jax.lax.Precision.HIGHEST)

    c = _D * _EPS * _EPS
    base = no_i + 2.0 * _EPS * so_i + c
    pos_sq = base + no_j - 2.0 * gp - 2.0 * _EPS * so_j
    neg_sq = base + na_j - 2.0 * gn - 2.0 * _EPS * sa_j
    d_pos = jnp.sqrt(jnp.maximum(pos_sq, 0.0))
    d_neg = jnp.sqrt(jnp.maximum(neg_sq, 0.0))
    hinge = jnp.maximum(d_pos - d_neg + _MARGIN, 0.0)

    m0 = (lc_ref[...] == 0).astype(jnp.float32)
    m1 = (lr_ref[...] == 1).astype(jnp.float32)
    w = adj_ref[...] * m0 * m1
    partial = jnp.sum(w * hinge)

    @pl.when(i == 0)
    def _():
        out_ref[0, 0] = 0.0

    out_ref[0, 0] += partial


def kernel(orig, aug, l, adj):
    lc = l.reshape(_N, 1)
    lr = l.reshape(1, _N)
    out = pl.pallas_call(
        _loss_body,
        grid=(_N // _TR,),
        in_specs=[
            pl.BlockSpec((_N, _D), lambda i: (0, 0)),
            pl.BlockSpec((_TR, _D), lambda i: (i, 0)),
            pl.BlockSpec((_N, _D), lambda i: (0, 0)),
            pl.BlockSpec((_TR, 1), lambda i: (i, 0)),
            pl.BlockSpec((1, _N), lambda i: (0, 0)),
            pl.BlockSpec((_TR, _N), lambda i: (i, 0)),
        ],
        out_specs=pl.BlockSpec(memory_space=pltpu.SMEM),
        out_shape=jax.ShapeDtypeStruct((1, 1), jnp.float32),
        compiler_params=pltpu.CompilerParams(
            dimension_semantics=("arbitrary",)),
    )(orig, orig, aug, lc, lr, adj)
    return out[0, 0]


# speedup vs baseline: 220.0685x; 1.8975x over previous
import jax
import jax.numpy as jnp
from jax.experimental import pallas as pl
from jax.experimental.pallas import tpu as pltpu

_N, _D = 2048, 128
_TR = 256
_MARGIN = 1.0
_EPS = 1e-6


def _loss_body(orig_ref, anch_ref, aug_ref, lc_ref, lr_ref, adj_ref, out_ref,
               obf_ref, gbf_ref, rp_ref, rn_ref):
    i = pl.program_id(0)
    dn = (((1,), (1,)), ((), ()))

    @pl.when(i == 0)
    def _():
        o = orig_ref[...]
        g = aug_ref[...]
        obf_ref[...] = o.astype(jnp.bfloat16)
        gbf_ref[...] = g.astype(jnp.bfloat16)
        ones = jnp.ones((1, _D), jnp.float32)
        no_j = jax.lax.dot_general(ones, o * o, dn,
                                   preferred_element_type=jnp.float32)
        so_j = jax.lax.dot_general(ones, o, dn,
                                   preferred_element_type=jnp.float32)
        na_j = jax.lax.dot_general(ones, g * g, dn,
                                   preferred_element_type=jnp.float32)
        sa_j = jax.lax.dot_general(ones, g, dn,
                                   preferred_element_type=jnp.float32)
        rp_ref[...] = no_j - (2.0 * _EPS) * so_j
        rn_ref[...] = na_j - (2.0 * _EPS) * sa_j
        out_ref[0, 0] = 0.0

    a = anch_ref[...]
    base = (jnp.sum(a * a, axis=1, keepdims=True)
            + (2.0 * _EPS) * jnp.sum(a, axis=1, keepdims=True)
            + _D * _EPS * _EPS)
    a2 = (a * -2.0).astype(jnp.bfloat16)

    gp = jax.lax.dot_general(a2, obf_ref[...], dn,
                             preferred_element_type=jnp.float32)
    gn = jax.lax.dot_general(a2, gbf_ref[...], dn,
                             preferred_element_type=jnp.float32)

    pos_sq = (gp + base) + rp_ref[...]
    neg_sq = (gn + base) + rn_ref[...]
    d_pos = jnp.sqrt(jnp.maximum(pos_sq, 0.0))
    d_neg = jnp.sqrt(jnp.maximum(neg_sq, 0.0))
    hinge = jnp.maximum(d_pos - d_neg + _MARGIN, 0.0)

    m1 = (lr_ref[...] == 1).astype(jnp.float32)
    am = adj_ref[...] * m1
    rowsum = jnp.sum(am * hinge, axis=1, keepdims=True)
    m0 = (lc_ref[...] == 0).astype(jnp.float32)
    out_ref[0, 0] += jnp.sum(rowsum * m0)


def kernel(orig, aug, l, adj):
    lc = l.reshape(_N, 1)
    lr = l.reshape(1, _N)
    out = pl.pallas_call(
        _loss_body,
        grid=(_N // _TR,),
        in_specs=[
            pl.BlockSpec((_N, _D), lambda i: (0, 0)),
            pl.BlockSpec((_TR, _D), lambda i: (i, 0)),
            pl.BlockSpec((_N, _D), lambda i: (0, 0)),
            pl.BlockSpec((_TR, 1), lambda i: (i, 0)),
            pl.BlockSpec((1, _N), lambda i: (0, 0)),
            pl.BlockSpec((_TR, _N), lambda i: (i, 0)),
        ],
        out_specs=pl.BlockSpec(memory_space=pltpu.SMEM),
        out_shape=jax.ShapeDtypeStruct((1, 1), jnp.float32),
        scratch_shapes=[
            pltpu.VMEM((_N, _D), jnp.bfloat16),
            pltpu.VMEM((_N, _D), jnp.bfloat16),
            pltpu.VMEM((1, _N), jnp.float32),
            pltpu.VMEM((1, _N), jnp.float32),
        ],
        compiler_params=pltpu.CompilerParams(
            dimension_semantics=("arbitrary",)),
    )(orig, orig, aug, lc, lr, adj)
    return out[0, 0]


# speedup vs baseline: 236.4848x; 1.0746x over previous
import jax
import jax.numpy as jnp
from jax.experimental import pallas as pl
from jax.experimental.pallas import tpu as pltpu

_N, _D = 2048, 128
_TR = 256
_K = 256
_MARGIN = 1.0
_EPS = 1e-6
_BIG = 1e6


def _loss_body(orig_ref, anch_ref, aug_ref, l_ref, adj_ref, out_ref,
               a_ref, bp_ref, bn_ref):
    i = pl.program_id(0)
    dn = (((1,), (1,)), ((), ()))

    @pl.when(i == 0)
    def _():
        o = orig_ref[...]
        g = aug_ref[...]
        lv = l_ref[...]
        m1c = (lv != 1).astype(jnp.float32)
        rp = (jnp.sum(o * o, axis=1, keepdims=True)
              - (2.0 * _EPS) * jnp.sum(o, axis=1, keepdims=True))
        rn = (jnp.sum(g * g, axis=1, keepdims=True)
              - (2.0 * _EPS) * jnp.sum(g, axis=1, keepdims=True)
              + _BIG * m1c)
        bp_ref[...] = jnp.zeros((_N, _K), jnp.bfloat16)
        bn_ref[...] = jnp.zeros((_N, _K), jnp.bfloat16)
        a_ref[...] = jnp.zeros((_TR, _K), jnp.bfloat16)
        ones_col = jnp.ones((_N, 1), jnp.bfloat16)
        bp_ref[:, 0:_D] = o.astype(jnp.bfloat16)
        bp_ref[:, _D:_D + 1] = ones_col
        bp_ref[:, _D + 1:_D + 2] = rp.astype(jnp.bfloat16)
        bn_ref[:, 0:_D] = g.astype(jnp.bfloat16)
        bn_ref[:, _D:_D + 1] = ones_col
        bn_ref[:, _D + 1:_D + 2] = rn.astype(jnp.bfloat16)
        bn_ref[:, _D + 2:_D + 3] = ones_col
        out_ref[0, 0] = 0.0

    a = anch_ref[...]
    base = (jnp.sum(a * a, axis=1, keepdims=True)
            + (2.0 * _EPS) * jnp.sum(a, axis=1, keepdims=True)
            + _D * _EPS * _EPS)
    lt = l_ref[pl.ds(i * _TR, _TR), :]
    big_m0 = _BIG * (lt != 0).astype(jnp.float32)

    a_ref[:, 0:_D] = (a * -2.0).astype(jnp.bfloat16)
    a_ref[:, _D:_D + 1] = base.astype(jnp.bfloat16)
    a_ref[:, _D + 1:_D + 2] = jnp.ones((_TR, 1), jnp.bfloat16)
    a_ref[:, _D + 2:_D + 3] = big_m0.astype(jnp.bfloat16)

    av = a_ref[...]
    pos_sq = jax.lax.dot_general(av, bp_ref[...], dn,
                                 preferred_element_type=jnp.float32)
    neg_sq = jax.lax.dot_general(av, bn_ref[...], dn,
                                 preferred_element_type=jnp.float32)

    mp = jnp.maximum(pos_sq, 1e-12)
    mn = jnp.maximum(neg_sq, 1e-12)
    d_pos = mp * jax.lax.rsqrt(mp)
    d_neg = mn * jax.lax.rsqrt(mn)
    hinge = jnp.maximum(d_pos - d_neg + _MARGIN, 0.0)
    out_ref[0, 0] += jnp.sum(adj_ref[...] * hinge)


def kernel(orig, aug, l, adj):
    lc = l.reshape(_N, 1)
    out = pl.pallas_call(
        _loss_body,
        grid=(_N // _TR,),
        in_specs=[
            pl.BlockSpec((_N, _D), lambda i: (0, 0)),
            pl.BlockSpec((_TR, _D), lambda i: (i, 0)),
            pl.BlockSpec((_N, _D), lambda i: (0, 0)),
            pl.BlockSpec((_N, 1), lambda i: (0, 0)),
            pl.BlockSpec((_TR, _N), lambda i: (i, 0)),
        ],
        out_specs=pl.BlockSpec(memory_space=pltpu.SMEM),
        out_shape=jax.ShapeDtypeStruct((1, 1), jnp.float32),
        scratch_shapes=[
            pltpu.VMEM((_TR, _K), jnp.bfloat16),
            pltpu.VMEM((_N, _K), jnp.bfloat16),
            pltpu.VMEM((_N, _K), jnp.bfloat16),
        ],
        compiler_params=pltpu.CompilerParams(
            dimension_semantics=("arbitrary",)),
    )(orig, orig, aug, lc, adj)
    return out[0, 0]


# speedup vs baseline: 270.3595x; 1.1432x over previous
import jax
import jax.numpy as jnp
from jax.experimental import pallas as pl
from jax.experimental.pallas import tpu as pltpu

_N, _D = 2048, 128
_TR = 256
_K = 256
_NH = _N // 2
_MARGIN = 1.0
_EPS = 1e-6
_BIG = 1e6


def _loss_body(orig_ref, aug_ref, l_ref, adjl_ref, adjr_ref, out_ref,
               af_ref, bp_ref, bn_ref):
    i = pl.program_id(0)
    dn = (((1,), (1,)), ((), ()))

    @pl.when(i == 0)
    def _():
        o = orig_ref[...]
        g = aug_ref[...]
        lv = l_ref[...]
        no = jnp.sum(o * o, axis=1, keepdims=True)
        so = jnp.sum(o, axis=1, keepdims=True)
        na = jnp.sum(g * g, axis=1, keepdims=True)
        sa = jnp.sum(g, axis=1, keepdims=True)
        rp = no - (2.0 * _EPS) * so
        rn = (na - (2.0 * _EPS) * sa
              + _BIG * (lv != 1).astype(jnp.float32))
        base = no + (2.0 * _EPS) * so + _D * _EPS * _EPS
        big_m0 = _BIG * (lv != 0).astype(jnp.float32)
        ones_col = jnp.ones((_N, 1), jnp.bfloat16)

        af_ref[...] = jnp.zeros((_N, _K), jnp.bfloat16)
        af_ref[:, 0:_D] = (o * -2.0).astype(jnp.bfloat16)
        af_ref[:, _D:_D + 1] = base.astype(jnp.bfloat16)
        af_ref[:, _D + 1:_D + 2] = ones_col
        af_ref[:, _D + 2:_D + 3] = big_m0.astype(jnp.bfloat16)

        bp_ref[...] = jnp.zeros((_N, _K), jnp.bfloat16)
        bp_ref[:, 0:_D] = o.astype(jnp.bfloat16)
        bp_ref[:, _D:_D + 1] = ones_col
        bp_ref[:, _D + 1:_D + 2] = rp.astype(jnp.bfloat16)

        bn_ref[...] = jnp.zeros((_N, _K), jnp.bfloat16)
        bn_ref[:, 0:_D] = g.astype(jnp.bfloat16)
        bn_ref[:, _D:_D + 1] = ones_col
        bn_ref[:, _D + 1:_D + 2] = rn.astype(jnp.bfloat16)
        bn_ref[:, _D + 2:_D + 3] = ones_col
        out_ref[0, 0] = 0.0

    av = af_ref[pl.ds(i * _TR, _TR), :]
    pos_sq = jax.lax.dot_general(av, bp_ref[...], dn,
                                 preferred_element_type=jnp.float32)
    neg_sq = jax.lax.dot_general(av, bn_ref[...], dn,
                                 preferred_element_type=jnp.float32)

    mp = jnp.maximum(pos_sq, 1e-12)
    mn = jnp.maximum(neg_sq, 1e-12)
    d_pos = mp * jax.lax.rsqrt(mp)
    d_neg = mn * jax.lax.rsqrt(mn)
    hinge = jnp.maximum(d_pos - d_neg + _MARGIN, 0.0)
    out_ref[0, 0] += (jnp.sum(adjl_ref[...] * hinge[:, 0:_NH])
                      + jnp.sum(adjr_ref[...] * hinge[:, _NH:_N]))


def kernel(orig, aug, l, adj):
    lc = l.reshape(_N, 1)
    out = pl.pallas_call(
        _loss_body,
        grid=(_N // _TR,),
        in_specs=[
            pl.BlockSpec((_N, _D), lambda i: (0, 0)),
            pl.BlockSpec((_N, _D), lambda i: (0, 0)),
            pl.BlockSpec((_N, 1), lambda i: (0, 0)),
            pl.BlockSpec((_TR, _NH), lambda i: (i, 0)),
            pl.BlockSpec((_TR, _NH), lambda i: (i, 1)),
        ],
        out_specs=pl.BlockSpec(memory_space=pltpu.SMEM),
        out_shape=jax.ShapeDtypeStruct((1, 1), jnp.float32),
        scratch_shapes=[
            pltpu.VMEM((_N, _K), jnp.bfloat16),
            pltpu.VMEM((_N, _K), jnp.bfloat16),
            pltpu.VMEM((_N, _K), jnp.bfloat16),
        ],
        compiler_params=pltpu.CompilerParams(
            dimension_semantics=("arbitrary",)),
    )(orig, aug, lc, adj, adj)
    return out[0, 0]


# speedup vs baseline: 285.8269x; 1.0572x over previous
import jax
import jax.numpy as jnp
from jax.experimental import pallas as pl
from jax.experimental.pallas import tpu as pltpu

_N, _D = 2048, 128
_TR = 256
_K = 256
_NQ = _N // 4
_MARGIN = 1.0
_EPS = 1e-6
_BIG = 1e6


def _loss_body(orig_ref, aug_ref, l_ref, adj0_ref, adj1_ref, adj2_ref,
               adj3_ref, out_ref, af_ref, bp_ref, bn_ref):
    i = pl.program_id(0)
    dn = (((1,), (1,)), ((), ()))

    @pl.when(i == 0)
    def _():
        o = orig_ref[...]
        g = aug_ref[...]
        lv = l_ref[...]
        no = jnp.sum(o * o, axis=1, keepdims=True)
        so = jnp.sum(o, axis=1, keepdims=True)
        na = jnp.sum(g * g, axis=1, keepdims=True)
        sa = jnp.sum(g, axis=1, keepdims=True)
        rp = no - (2.0 * _EPS) * so
        rn = (na - (2.0 * _EPS) * sa
              + _BIG * (lv != 1).astype(jnp.float32))
        base = no + (2.0 * _EPS) * so + _D * _EPS * _EPS
        big_m0 = _BIG * (lv != 0).astype(jnp.float32)
        ones_col = jnp.ones((_N, 1), jnp.bfloat16)

        af_ref[...] = jnp.zeros((_N, _K), jnp.bfloat16)
        af_ref[:, 0:_D] = (o * -2.0).astype(jnp.bfloat16)
        af_ref[:, _D:_D + 1] = base.astype(jnp.bfloat16)
        af_ref[:, _D + 1:_D + 2] = ones_col
        af_ref[:, _D + 2:_D + 3] = big_m0.astype(jnp.bfloat16)

        bp_ref[...] = jnp.zeros((_N, _K), jnp.bfloat16)
        bp_ref[:, 0:_D] = o.astype(jnp.bfloat16)
        bp_ref[:, _D:_D + 1] = ones_col
        bp_ref[:, _D + 1:_D + 2] = rp.astype(jnp.bfloat16)

        bn_ref[...] = jnp.zeros((_N, _K), jnp.bfloat16)
        bn_ref[:, 0:_D] = g.astype(jnp.bfloat16)
        bn_ref[:, _D:_D + 1] = ones_col
        bn_ref[:, _D + 1:_D + 2] = rn.astype(jnp.bfloat16)
        bn_ref[:, _D + 2:_D + 3] = ones_col
        out_ref[0, 0] = 0.0

    av = af_ref[pl.ds(i * _TR, _TR), :]
    pos_sq = jax.lax.dot_general(av, bp_ref[...], dn,
                                 preferred_element_type=jnp.float32)
    neg_sq = jax.lax.dot_general(av, bn_ref[...], dn,
                                 preferred_element_type=jnp.float32)

    mp = jnp.maximum(pos_sq, 1e-12)
    mn = jnp.maximum(neg_sq, 1e-12)
    d_pos = mp * jax.lax.rsqrt(mp)
    d_neg = mn * jax.lax.rsqrt(mn)
    hinge = jnp.maximum(d_pos - d_neg + _MARGIN, 0.0)
    out_ref[0, 0] += (
        jnp.sum(adj0_ref[...] * hinge[:, 0 * _NQ:1 * _NQ])
        + jnp.sum(adj1_ref[...] * hinge[:, 1 * _NQ:2 * _NQ])
        + jnp.sum(adj2_ref[...] * hinge[:, 2 * _NQ:3 * _NQ])
        + jnp.sum(adj3_ref[...] * hinge[:, 3 * _NQ:4 * _NQ]))


def kernel(orig, aug, l, adj):
    lc = l.reshape(_N, 1)
    out = pl.pallas_call(
        _loss_body,
        grid=(_N // _TR,),
        in_specs=[
            pl.BlockSpec((_N, _D), lambda i: (0, 0)),
            pl.BlockSpec((_N, _D), lambda i: (0, 0)),
            pl.BlockSpec((_N, 1), lambda i: (0, 0)),
            pl.BlockSpec((_TR, _NQ), lambda i: (i, 0)),
            pl.BlockSpec((_TR, _NQ), lambda i: (i, 1)),
            pl.BlockSpec((_TR, _NQ), lambda i: (i, 2)),
            pl.BlockSpec((_TR, _NQ), lambda i: (i, 3)),
        ],
        out_specs=pl.BlockSpec(memory_space=pltpu.SMEM),
        out_shape=jax.ShapeDtypeStruct((1, 1), jnp.float32),
        scratch_shapes=[
            pltpu.VMEM((_N, _K), jnp.bfloat16),
            pltpu.VMEM((_N, _K), jnp.bfloat16),
            pltpu.VMEM((_N, _K), jnp.bfloat16),
        ],
        compiler_params=pltpu.CompilerParams(
            dimension_semantics=("arbitrary",)),
    )(orig, aug, lc, adj, adj, adj, adj)
    return out[0, 0]


# speedup vs baseline: 292.7410x; 1.0242x over previous
import jax
import jax.numpy as jnp
from jax.experimental import pallas as pl
from jax.experimental.pallas import tpu as pltpu

_N, _D = 2048, 128
_TR = 512
_K = 256
_NQ = _N // 4
_MARGIN = 1.0
_EPS = 1e-6
_BIG = 1e6


def _loss_body(orig_ref, aug_ref, l_ref, adj0_ref, adj1_ref, adj2_ref,
               adj3_ref, out_ref, af_ref, bp_ref, bn_ref):
    i = pl.program_id(0)
    dn = (((1,), (1,)), ((), ()))

    @pl.when(i == 0)
    def _():
        o = orig_ref[...]
        g = aug_ref[...]
        lv = l_ref[...]
        no = jnp.sum(o * o, axis=1, keepdims=True)
        so = jnp.sum(o, axis=1, keepdims=True)
        na = jnp.sum(g * g, axis=1, keepdims=True)
        sa = jnp.sum(g, axis=1, keepdims=True)
        rp = no - (2.0 * _EPS) * so
        rn = (na - (2.0 * _EPS) * sa
              + _BIG * (lv != 1).astype(jnp.float32))
        base = no + (2.0 * _EPS) * so + _D * _EPS * _EPS
        big_m0 = _BIG * (lv != 0).astype(jnp.float32)
        ones_col = jnp.ones((_N, 1), jnp.bfloat16)

        af_ref[...] = jnp.zeros((_N, _K), jnp.bfloat16)
        af_ref[:, 0:_D] = (o * -2.0).astype(jnp.bfloat16)
        af_ref[:, _D:_D + 1] = base.astype(jnp.bfloat16)
        af_ref[:, _D + 1:_D + 2] = ones_col
        af_ref[:, _D + 2:_D + 3] = big_m0.astype(jnp.bfloat16)

        bp_ref[...] = jnp.zeros((_N, _K), jnp.bfloat16)
        bp_ref[:, 0:_D] = o.astype(jnp.bfloat16)
        bp_ref[:, _D:_D + 1] = ones_col
        bp_ref[:, _D + 1:_D + 2] = rp.astype(jnp.bfloat16)

        bn_ref[...] = jnp.zeros((_N, _K), jnp.bfloat16)
        bn_ref[:, 0:_D] = g.astype(jnp.bfloat16)
        bn_ref[:, _D:_D + 1] = ones_col
        bn_ref[:, _D + 1:_D + 2] = rn.astype(jnp.bfloat16)
        bn_ref[:, _D + 2:_D + 3] = ones_col
        out_ref[0, 0] = 0.0

    av = af_ref[pl.ds(i * _TR, _TR), :]
    pos_sq = jax.lax.dot_general(av, bp_ref[...], dn,
                                 preferred_element_type=jnp.float32)
    neg_sq = jax.lax.dot_general(av, bn_ref[...], dn,
                                 preferred_element_type=jnp.float32)

    mp = jnp.maximum(pos_sq, 1e-12)
    mn = jnp.maximum(neg_sq, 1e-12)
    d_pos = mp * jax.lax.rsqrt(mp)
    d_neg = mn * jax.lax.rsqrt(mn)
    hinge = jnp.maximum(d_pos - d_neg + _MARGIN, 0.0)
    out_ref[0, 0] += (
        jnp.sum(adj0_ref[...] * hinge[:, 0 * _NQ:1 * _NQ])
        + jnp.sum(adj1_ref[...] * hinge[:, 1 * _NQ:2 * _NQ])
        + jnp.sum(adj2_ref[...] * hinge[:, 2 * _NQ:3 * _NQ])
        + jnp.sum(adj3_ref[...] * hinge[:, 3 * _NQ:4 * _NQ]))


def kernel(orig, aug, l, adj):
    lc = l.reshape(_N, 1)
    out = pl.pallas_call(
        _loss_body,
        grid=(_N // _TR,),
        in_specs=[
            pl.BlockSpec((_N, _D), lambda i: (0, 0)),
            pl.BlockSpec((_N, _D), lambda i: (0, 0)),
            pl.BlockSpec((_N, 1), lambda i: (0, 0)),
            pl.BlockSpec((_TR, _NQ), lambda i: (i, 0)),
            pl.BlockSpec((_TR, _NQ), lambda i: (i, 1)),
            pl.BlockSpec((_TR, _NQ), lambda i: (i, 2)),
            pl.BlockSpec((_TR, _NQ), lambda i: (i, 3)),
        ],
        out_specs=pl.BlockSpec(memory_space=pltpu.SMEM),
        out_shape=jax.ShapeDtypeStruct((1, 1), jnp.float32),
        scratch_shapes=[
            pltpu.VMEM((_N, _K), jnp.bfloat16),
            pltpu.VMEM((_N, _K), jnp.bfloat16),
            pltpu.VMEM((_N, _K), jnp.bfloat16),
        ],
        compiler_params=pltpu.CompilerParams(
            dimension_semantics=("arbitrary",)),
    )(orig, aug, lc, adj, adj, adj, adj)
    return out[0, 0]


# speedup vs baseline: 295.8423x; 1.0106x over previous
import jax
import jax.numpy as jnp
from jax.experimental import pallas as pl
from jax.experimental.pallas import tpu as pltpu

_N, _D = 2048, 128
_TR = 512
_K = 256
_NQ = _N // 8
_MARGIN = 1.0
_EPS = 1e-6
_BIG = 1e6


def _loss_body(orig_ref, aug_ref, l_ref, adj0_ref, adj1_ref, adj2_ref,
               adj3_ref, adj4_ref, adj5_ref, adj6_ref, adj7_ref, out_ref,
               af_ref, bp_ref, bn_ref):
    i = pl.program_id(0)
    dn = (((1,), (1,)), ((), ()))

    @pl.when(i == 0)
    def _():
        o = orig_ref[...]
        g = aug_ref[...]
        lv = l_ref[...]
        no = jnp.sum(o * o, axis=1, keepdims=True)
        so = jnp.sum(o, axis=1, keepdims=True)
        na = jnp.sum(g * g, axis=1, keepdims=True)
        sa = jnp.sum(g, axis=1, keepdims=True)
        rp = no - (2.0 * _EPS) * so
        rn = (na - (2.0 * _EPS) * sa
              + _BIG * (lv != 1).astype(jnp.float32))
        base = no + (2.0 * _EPS) * so + _D * _EPS * _EPS
        big_m0 = _BIG * (lv != 0).astype(jnp.float32)
        ones_col = jnp.ones((_N, 1), jnp.bfloat16)

        af_ref[...] = jnp.zeros((_N, _K), jnp.bfloat16)
        af_ref[:, 0:_D] = (o * -2.0).astype(jnp.bfloat16)
        af_ref[:, _D:_D + 1] = base.astype(jnp.bfloat16)
        af_ref[:, _D + 1:_D + 2] = ones_col
        af_ref[:, _D + 2:_D + 3] = big_m0.astype(jnp.bfloat16)

        bp_ref[...] = jnp.zeros((_N, _K), jnp.bfloat16)
        bp_ref[:, 0:_D] = o.astype(jnp.bfloat16)
        bp_ref[:, _D:_D + 1] = ones_col
        bp_ref[:, _D + 1:_D + 2] = rp.astype(jnp.bfloat16)

        bn_ref[...] = jnp.zeros((_N, _K), jnp.bfloat16)
        bn_ref[:, 0:_D] = g.astype(jnp.bfloat16)
        bn_ref[:, _D:_D + 1] = ones_col
        bn_ref[:, _D + 1:_D + 2] = rn.astype(jnp.bfloat16)
        bn_ref[:, _D + 2:_D + 3] = ones_col
        out_ref[0, 0] = 0.0

    av = af_ref[pl.ds(i * _TR, _TR), :]
    pos_sq = jax.lax.dot_general(av, bp_ref[...], dn,
                                 preferred_element_type=jnp.float32)
    neg_sq = jax.lax.dot_general(av, bn_ref[...], dn,
                                 preferred_element_type=jnp.float32)

    mp = jnp.maximum(pos_sq, 1e-12)
    mn = jnp.maximum(neg_sq, 1e-12)
    d_pos = mp * jax.lax.rsqrt(mp)
    d_neg = mn * jax.lax.rsqrt(mn)
    hinge = jnp.maximum(d_pos - d_neg + _MARGIN, 0.0)
    out_ref[0, 0] += (
        jnp.sum(adj0_ref[...] * hinge[:, 0 * _NQ:1 * _NQ])
        + jnp.sum(adj1_ref[...] * hinge[:, 1 * _NQ:2 * _NQ])
        + jnp.sum(adj2_ref[...] * hinge[:, 2 * _NQ:3 * _NQ])
        + jnp.sum(adj3_ref[...] * hinge[:, 3 * _NQ:4 * _NQ])
        + jnp.sum(adj4_ref[...] * hinge[:, 4 * _NQ:5 * _NQ])
        + jnp.sum(adj5_ref[...] * hinge[:, 5 * _NQ:6 * _NQ])
        + jnp.sum(adj6_ref[...] * hinge[:, 6 * _NQ:7 * _NQ])
        + jnp.sum(adj7_ref[...] * hinge[:, 7 * _NQ:8 * _NQ]))


def kernel(orig, aug, l, adj):
    lc = l.reshape(_N, 1)
    out = pl.pallas_call(
        _loss_body,
        grid=(_N // _TR,),
        in_specs=[
            pl.BlockSpec((_N, _D), lambda i: (0, 0)),
            pl.BlockSpec((_N, _D), lambda i: (0, 0)),
            pl.BlockSpec((_N, 1), lambda i: (0, 0)),
            pl.BlockSpec((_TR, _NQ), lambda i: (i, 0)),
            pl.BlockSpec((_TR, _NQ), lambda i: (i, 1)),
            pl.BlockSpec((_TR, _NQ), lambda i: (i, 2)),
            pl.BlockSpec((_TR, _NQ), lambda i: (i, 3)),
            pl.BlockSpec((_TR, _NQ), lambda i: (i, 4)),
            pl.BlockSpec((_TR, _NQ), lambda i: (i, 5)),
            pl.BlockSpec((_TR, _NQ), lambda i: (i, 6)),
            pl.BlockSpec((_TR, _NQ), lambda i: (i, 7)),
        ],
        out_specs=pl.BlockSpec(memory_space=pltpu.SMEM),
        out_shape=jax.ShapeDtypeStruct((1, 1), jnp.float32),
        scratch_shapes=[
            pltpu.VMEM((_N, _K), jnp.bfloat16),
            pltpu.VMEM((_N, _K), jnp.bfloat16),
            pltpu.VMEM((_N, _K), jnp.bfloat16),
        ],
        compiler_params=pltpu.CompilerParams(
            dimension_semantics=("arbitrary",)),
    )(orig, aug, lc, adj, adj, adj, adj, adj, adj, adj, adj)
    return out[0, 0]
